# Initial kernel scaffold; baseline (speedup 1.0000x reference)
import jax
import jax.numpy as jnp
from jax.experimental import pallas as pl


def kernel(x_feat, z_feat, emb, skip0, skip1, params, idx_up0, idx_up1, idx_p2v):
    raise NotImplementedError("write your pallas kernel here")


# speedup vs baseline: 1.4983x; 1.4983x over previous
import functools

import jax
import jax.numpy as jnp
from jax import lax
from jax.experimental import pallas as pl
from jax.experimental.pallas import tpu as pltpu
from jax.experimental.pallas import tpu_sc as plsc

F32 = jnp.float32
I32 = jnp.int32
EPS = 1e-5

NV0, NV1, NV2 = 25000, 50000, 100000
NP_ = 100000
NC, NS, L = 2, 16, 16
NW = NC * NS
CH = 128

NV1P = 53248
NPP = 102400
HALF = NPP // 2
ACC_R = 51456
DUMP = HALF + 64


def _silu(x):
    return x * jax.nn.sigmoid(x)


def _bn_affine(stats, g, b, n):
    m = stats[0:1, :] / n
    v = stats[1:2, :] / n - m * m
    scale = g / jnp.sqrt(v + EPS)
    return scale, b - m * scale


def _stats2(a, b, n, rb):
    grid = n // rb
    ca, cb = a.shape[1], b.shape[1]

    def body(a_ref, b_ref, oa_ref, ob_ref):
        i = pl.program_id(0)
        for x_ref, o_ref, c in ((a_ref, oa_ref, ca), (b_ref, ob_ref, cb)):
            x = x_ref[...]
            s = jnp.sum(x, axis=0)
            q = jnp.sum(x * x, axis=0)
            acc = jnp.concatenate(
                [s[None], q[None], jnp.zeros((6, c), F32)], axis=0)

            @pl.when(i == 0)
            def _():
                o_ref[...] = acc

            @pl.when(i > 0)
            def _():
                o_ref[...] = o_ref[...] + acc

    return pl.pallas_call(
        body,
        grid=(grid,),
        in_specs=[pl.BlockSpec((rb, ca), lambda i: (i, 0)),
                  pl.BlockSpec((rb, cb), lambda i: (i, 0))],
        out_specs=[pl.BlockSpec((8, ca), lambda i: (0, 0)),
                   pl.BlockSpec((8, cb), lambda i: (0, 0))],
        out_shape=[jax.ShapeDtypeStruct((8, ca), F32),
                   jax.ShapeDtypeStruct((8, cb), F32)],
    )(a, b)


def _stats1(a, n, rb):
    grid = n // rb
    ca = a.shape[1]

    def body(a_ref, oa_ref):
        i = pl.program_id(0)
        x = a_ref[...]
        s = jnp.sum(x, axis=0)
        q = jnp.sum(x * x, axis=0)
        acc = jnp.concatenate(
            [s[None], q[None], jnp.zeros((6, ca), F32)], axis=0)

        @pl.when(i == 0)
        def _():
            oa_ref[...] = acc

        @pl.when(i > 0)
        def _():
            oa_ref[...] = oa_ref[...] + acc

    return pl.pallas_call(
        body,
        grid=(grid,),
        in_specs=[pl.BlockSpec((rb, ca), lambda i: (i, 0))],
        out_specs=[pl.BlockSpec((8, ca), lambda i: (0, 0))],
        out_shape=[jax.ShapeDtypeStruct((8, ca), F32)],
    )(a)[0]


def _res_half1(a, b, sa, sb, emb, prm, ca, cb, nf, n, rb):
    grid = n // rb
    g1a = prm['g1'][:ca].reshape(1, ca)
    g1b = prm['g1'][ca:].reshape(1, cb)
    b1a = prm['b1'][:ca].reshape(1, ca)
    b1b = prm['b1'][ca:].reshape(1, cb)
    w1a = prm['W1'][:ca]
    w1b = prm['W1'][ca:]
    c1 = prm['c1'].reshape(1, nf)
    ce = prm['ce'].reshape(1, nf)

    def body(a_ref, b_ref, sa_ref, sb_ref, emb_ref, we_ref, g1a_ref,
             b1a_ref, g1b_ref, b1b_ref, w1a_ref, w1b_ref, c1_ref, ce_ref,
             mid_ref, ms_ref):
        i = pl.program_id(0)
        sca, ta = _bn_affine(sa_ref[...], g1a_ref[...], b1a_ref[...], n)
        scb, tb = _bn_affine(sb_ref[...], g1b_ref[...], b1b_ref[...], n)
        xa = _silu(a_ref[...] * sca + ta)
        xb = _silu(b_ref[...] * scb + tb)
        et = jnp.dot(_silu(emb_ref[...]), we_ref[...],
                     preferred_element_type=F32) + ce_ref[...]
        h = (jnp.dot(xa, w1a_ref[...], preferred_element_type=F32)
             + jnp.dot(xb, w1b_ref[...], preferred_element_type=F32)
             + c1_ref[...] + et)
        mid_ref[...] = h
        s = jnp.sum(h, axis=0)
        q = jnp.sum(h * h, axis=0)
        acc = jnp.concatenate([s[None], q[None], jnp.zeros((6, nf), F32)],
                              axis=0)

        @pl.when(i == 0)
        def _():
            ms_ref[...] = acc

        @pl.when(i > 0)
        def _():
            ms_ref[...] = ms_ref[...] + acc

    full = lambda r, c: pl.BlockSpec((r, c), lambda i: (0, 0))
    return pl.pallas_call(
        body,
        grid=(grid,),
        in_specs=[pl.BlockSpec((rb, ca), lambda i: (i, 0)),
                  pl.BlockSpec((rb, cb), lambda i: (i, 0)),
                  full(8, ca), full(8, cb), full(1, 512), full(512, nf),
                  full(1, ca), full(1, ca), full(1, cb), full(1, cb),
                  full(ca, nf), full(cb, nf), full(1, nf), full(1, nf)],
        out_specs=[pl.BlockSpec((rb, nf), lambda i: (i, 0)),
                   pl.BlockSpec((8, nf), lambda i: (0, 0))],
        out_shape=[jax.ShapeDtypeStruct((n, nf), F32),
                   jax.ShapeDtypeStruct((8, nf), F32)],
    )(a, b, sa, sb, emb, prm['We'], g1a, b1a, g1b, b1b, w1a, w1b, c1, ce)


def _res_half2(mid, ms, a, b, prm, up_w, up_b, ca, cb, nf, fo, n, rb):
    grid = n // rb
    g2 = prm['g2'].reshape(1, nf)
    b2 = prm['b2'].reshape(1, nf)
    c2 = prm['c2'].reshape(1, nf)
    wsa = prm['Ws'][:ca]
    wsb = prm['Ws'][ca:]
    upb = up_b.reshape(1, fo)

    def body(mid_ref, ms_ref, a_ref, b_ref, g2_ref, b2_ref, w2_ref,
             c2_ref, wsa_ref, wsb_ref, upw_ref, upb_ref, u_ref):
        sc, t = _bn_affine(ms_ref[...], g2_ref[...], b2_ref[...], n)
        x = _silu(mid_ref[...] * sc + t)
        h = (jnp.dot(x, w2_ref[...], preferred_element_type=F32)
             + jnp.dot(a_ref[...], wsa_ref[...], preferred_element_type=F32)
             + jnp.dot(b_ref[...], wsb_ref[...], preferred_element_type=F32)
             + c2_ref[...])
        u_ref[...] = jnp.dot(h, upw_ref[...],
                             preferred_element_type=F32) + upb_ref[...]

    full = lambda r, c: pl.BlockSpec((r, c), lambda i: (0, 0))
    return pl.pallas_call(
        body,
        grid=(grid,),
        in_specs=[pl.BlockSpec((rb, nf), lambda i: (i, 0)),
                  full(8, nf),
                  pl.BlockSpec((rb, ca), lambda i: (i, 0)),
                  pl.BlockSpec((rb, cb), lambda i: (i, 0)),
                  full(1, nf), full(1, nf), full(nf, nf), full(1, nf),
                  full(ca, nf), full(cb, nf), full(nf, fo), full(1, fo)],
        out_specs=[pl.BlockSpec((rb, fo), lambda i: (i, 0))],
        out_shape=[jax.ShapeDtypeStruct((n, fo), F32)],
    )(mid, ms, a, b, g2, b2, prm['W2'], c2, wsa, wsb, up_w, upb)[0]


def _point_mlp(z, sz, g, b, w, c, n_stats, rb):
    n = z.shape[0]
    grid = n // rb
    cz, fo = w.shape
    g2 = g.reshape(1, cz)
    b2 = b.reshape(1, cz)
    c2 = c.reshape(1, fo)

    def body(z_ref, sz_ref, g_ref, b_ref, w_ref, c_ref, o_ref):
        sc, t = _bn_affine(sz_ref[...], g_ref[...], b_ref[...], n_stats)
        x = _silu(z_ref[...] * sc + t)
        o_ref[...] = jnp.dot(x, w_ref[...],
                             preferred_element_type=F32) + c_ref[...]

    full = lambda r, cc: pl.BlockSpec((r, cc), lambda i: (0, 0))
    return pl.pallas_call(
        body,
        grid=(grid,),
        in_specs=[pl.BlockSpec((rb, cz), lambda i: (i, 0)),
                  full(8, cz), full(1, cz), full(1, cz),
                  full(cz, fo), full(1, fo)],
        out_specs=[pl.BlockSpec((rb, fo), lambda i: (i, 0))],
        out_shape=[jax.ShapeDtypeStruct((n, fo), F32)],
    )(z, sz, g2, b2, w, c2)[0]


def _seg_divide(sums, cnts, n, rb):
    grid = n // rb
    fo = sums.shape[1]

    def body(s_ref, c_ref, o_ref):
        o_ref[...] = s_ref[...] / jnp.maximum(c_ref[...], 1.0)

    return pl.pallas_call(
        body,
        grid=(grid,),
        in_specs=[pl.BlockSpec((rb, fo), lambda i: (i, 0)),
                  pl.BlockSpec((rb, 1), lambda i: (i, 0))],
        out_specs=[pl.BlockSpec((rb, fo), lambda i: (i, 0))],
        out_shape=[jax.ShapeDtypeStruct((n, fo), F32)],
    )(sums, cnts)[0]


def _sc_mesh():
    return plsc.VectorSubcoreMesh(core_axis_name="c", subcore_axis_name="s")


def _gather_rows(table, idx, d):
    b = idx.shape[0]
    per_w = b // NW
    n_ch = per_w // CH

    @functools.partial(
        pl.kernel, mesh=_sc_mesh(),
        compiler_params=pltpu.CompilerParams(use_tc_tiling_on_sc=False, needs_layout_passes=False),
        out_type=jax.ShapeDtypeStruct((b, d), F32),
        scratch_types=[
            pltpu.VMEM((CH,), I32),
            pltpu.VMEM((CH, d), F32),
            pltpu.SemaphoreType.DMA,
        ],
    )
    def k(tab_hbm, idx_hbm, out_hbm, idx_v, rows_v, sem):
        wid = lax.axis_index("s") * NC + lax.axis_index("c")
        base = wid * per_w

        def chunk(j, _):
            off = base + j * CH
            pltpu.sync_copy(idx_hbm.at[pl.ds(off, CH)], idx_v)
            pltpu.async_copy(tab_hbm.at[idx_v], rows_v, sem).wait()
            pltpu.sync_copy(rows_v, out_hbm.at[pl.ds(off, CH)])
            return _

        lax.fori_loop(0, n_ch, chunk, 0)

    return k(table, idx)


def _compose_idx(idx_up1_p, idx_p2v_p):
    per_w = NPP // NW
    n_ch = per_w // CH

    @functools.partial(
        pl.kernel, mesh=_sc_mesh(),
        compiler_params=pltpu.CompilerParams(
            use_tc_tiling_on_sc=False, needs_layout_passes=False),
        out_type=jax.ShapeDtypeStruct((NPP,), I32),
        scratch_types=[
            pltpu.VMEM((NPP,), I32),
            pltpu.VMEM((CH,), I32),
            pltpu.VMEM((CH,), I32),
        ],
    )
    def k(tab_hbm, idxp_hbm, out_hbm, tab_v, idxp_v, idxc_v):
        wid = lax.axis_index("s") * NC + lax.axis_index("c")
        base = wid * per_w
        pltpu.sync_copy(tab_hbm, tab_v)

        def chunk(j, cr):
            off = base + j * CH
            pltpu.sync_copy(idxp_hbm.at[pl.ds(off, CH)], idxp_v)

            def grp(g, c2):
                iv = idxp_v[pl.ds(g * L, L)]
                idxc_v[pl.ds(g * L, L)] = plsc.load_gather(tab_v, [iv])
                return c2
            lax.fori_loop(0, CH // L, grp, 0)
            pltpu.sync_copy(idxc_v, out_hbm.at[pl.ds(off, CH)])
            return cr
        lax.fori_loop(0, n_ch, chunk, 0)

    return k(idx_up1_p, idx_p2v_p)


def _point_stage(u1, idx_p2v_p, idx_comp, pb):
    fo = u1.shape[1]
    n_ch = NPP // CH
    per_tile = n_ch // NS
    drain_ch = HALF // CH
    zero_ch = ACC_R // CH

    @functools.partial(
        pl.kernel, mesh=_sc_mesh(),
        compiler_params=pltpu.CompilerParams(use_tc_tiling_on_sc=False, needs_layout_passes=False),
        out_type=(jax.ShapeDtypeStruct((NPP, fo), F32),
                  jax.ShapeDtypeStruct((NPP, fo), F32),
                  jax.ShapeDtypeStruct((NPP,), F32)),
        scratch_types=[
            pltpu.VMEM_SHARED((ACC_R, fo), F32),
            pltpu.VMEM_SHARED((ACC_R,), F32),
            pltpu.VMEM((CH, fo), F32),
            pltpu.VMEM((CH, fo), F32),
            pltpu.VMEM((CH, fo), F32),
            pltpu.VMEM((CH,), F32),
            pltpu.VMEM((CH,), F32),
            pltpu.VMEM((CH,), I32),
            pltpu.VMEM((CH,), I32),
            pltpu.VMEM((CH,), I32),
            pltpu.SemaphoreType.DMA,
        ],
    )
    def k(u1_hbm, idxp_hbm, idxc_hbm, pb_hbm,
          z1_hbm, sums_hbm, cnts_hbm,
          acc, acc1, zbuf, pbbuf, cbuf, cbuf1, ones_v,
          idxp_v, idxc_v, sidx_v, sem):
        c = lax.axis_index("c")
        s = lax.axis_index("s")
        lo = c * HALF

        def z2(j, _):
            cbuf[j, pl.ds(0, L)] = jnp.zeros((L,), F32)
            cbuf[j, pl.ds(L, L)] = jnp.zeros((L,), F32)
            return _
        lax.fori_loop(0, CH, z2, 0)

        def z1i(g, _):
            cbuf1[pl.ds(g * L, L)] = jnp.zeros((L,), F32)
            ones_v[pl.ds(g * L, L)] = jnp.ones((L,), F32)
            return _
        lax.fori_loop(0, CH // L, z1i, 0)

        def zacc(j, cr):
            ch = s + j * NS

            @pl.when(ch < zero_ch)
            def _w():
                pltpu.sync_copy(cbuf, acc.at[pl.ds(ch * CH, CH)])
                pltpu.sync_copy(cbuf1, acc1.at[pl.ds(ch * CH, CH)])
            return cr
        lax.fori_loop(0, (zero_ch + NS - 1) // NS, zacc, 0)

        plsc.subcore_barrier()

        def chunk(j, cr):
            ch = s * per_tile + j
            base = ch * CH
            pltpu.sync_copy(idxp_hbm.at[pl.ds(base, CH)], idxp_v)
            pltpu.sync_copy(idxc_hbm.at[pl.ds(base, CH)], idxc_v)

            def grp(g, _2):
                iv = idxp_v[pl.ds(g * L, L)]
                inr = jnp.logical_and(iv >= lo, iv < lo + HALF)
                sidx_v[pl.ds(g * L, L)] = jnp.where(inr, iv - lo, DUMP)
                return _2
            lax.fori_loop(0, CH // L, grp, 0)

            pltpu.async_copy(u1_hbm.at[idxc_v], zbuf, sem).wait()
            pltpu.sync_copy(pb_hbm.at[pl.ds(base, CH)], pbbuf)

            def add(r, _2):
                zbuf[r, pl.ds(0, L)] = (zbuf[r, pl.ds(0, L)]
                                        + pbbuf[r, pl.ds(0, L)])
                zbuf[r, pl.ds(L, L)] = (zbuf[r, pl.ds(L, L)]
                                        + pbbuf[r, pl.ds(L, L)])
                return _2
            lax.fori_loop(0, CH, add, 0)

            @pl.when(lax.rem(ch, 2) == c)
            def _w():
                pltpu.sync_copy(zbuf, z1_hbm.at[pl.ds(base, CH)])

            pltpu.sync_copy(zbuf, acc.at[sidx_v], add=True)
            pltpu.sync_copy(ones_v, acc1.at[sidx_v], add=True)
            return cr
        lax.fori_loop(0, per_tile, chunk, 0)

        plsc.subcore_barrier()

        def drain(j, _):
            ch = s + j * NS
            row = ch * CH
            pltpu.sync_copy(acc.at[pl.ds(row, CH)], cbuf)
            pltpu.sync_copy(cbuf, sums_hbm.at[pl.ds(lo + row, CH)])
            pltpu.sync_copy(acc1.at[pl.ds(row, CH)], cbuf1)
            pltpu.sync_copy(cbuf1, cnts_hbm.at[pl.ds(lo + row, CH)])
            return _
        lax.fori_loop(0, drain_ch // NS, drain, 0)

    return k(u1, idx_p2v_p, idx_comp, pb)


def kernel(x_feat, z_feat, emb, skip0, skip1, params, idx_up0, idx_up1,
           idx_p2v):
    p = params
    idx_up0_p = jnp.concatenate(
        [idx_up0.astype(I32), jnp.zeros((NV1P - NV1,), I32)])
    idx_up1_p = jnp.concatenate(
        [idx_up1.astype(I32), jnp.zeros((NPP - NV2,), I32)])
    idx_p2v_p = jnp.concatenate(
        [idx_p2v.astype(I32), jnp.full((NPP - NP_,), NV2, I32)])
    z_feat_p = jnp.concatenate(
        [z_feat, jnp.zeros((NPP - NP_, z_feat.shape[1]), F32)])

    sa0, sb0 = _stats2(x_feat, skip0, NV0, 5000)
    mid0, ms0 = _res_half1(x_feat, skip0, sa0, sb0, emb, p['res0'],
                           128, 64, 128, NV0, 5000)
    u0 = _res_half2(mid0, ms0, x_feat, skip0, p['res0'],
                    p['up0_W'], p['up0_b'], 128, 64, 128, 64, NV0, 5000)
    x1 = _gather_rows(u0, idx_up0_p, 64)

    sa1, sb1 = _stats2(x1, skip1, NV1, 5000)
    mid1, ms1 = _res_half1(x1, skip1, sa1, sb1, emb, p['res1'],
                           64, 32, 64, NV1, 5000)
    u1 = _res_half2(mid1, ms1, x1, skip1, p['res1'],
                    p['up1_W'], p['up1_b'], 64, 32, 64, 32, NV1, 5000)

    sz = _stats1(z_feat, NP_, 5000)
    pb = _point_mlp(z_feat_p, sz, p['pb_g'], p['pb_b'], p['pb_W'],
                    p['pb_c'], NP_, 5120)

    idx_comp = _compose_idx(idx_up1_p, idx_p2v_p)
    z1p, sums, cnts = _point_stage(u1, idx_p2v_p, idx_comp, pb)

    x_out = _seg_divide(sums, cnts.reshape(NPP, 1), NV2, 5000)
    return (x_out, z1p[:NP_])


# speedup vs baseline: 1.6635x; 1.1103x over previous
import functools

import jax
import jax.numpy as jnp
from jax import lax
from jax.experimental import pallas as pl
from jax.experimental.pallas import tpu as pltpu
from jax.experimental.pallas import tpu_sc as plsc

F32 = jnp.float32
I32 = jnp.int32
EPS = 1e-5

NV0, NV1, NV2 = 25000, 50000, 100000
NP_ = 100000
NC, NS, L = 2, 16, 16
NW = NC * NS
CH = 128
KB = 5
KCH = KB * CH

NV1P = 53248
NPP = 102400
HALF = NPP // 2
ACC_R = 51456
DUMP = HALF + 64


def _silu(x):
    return x * jax.nn.sigmoid(x)


def _bn_affine(stats, g, b, n):
    m = stats[0:1, :] / n
    v = stats[1:2, :] / n - m * m
    scale = g / jnp.sqrt(v + EPS)
    return scale, b - m * scale


def _stats2(a, b, n, rb):
    grid = n // rb
    ca, cb = a.shape[1], b.shape[1]

    def body(a_ref, b_ref, oa_ref, ob_ref):
        i = pl.program_id(0)
        for x_ref, o_ref, c in ((a_ref, oa_ref, ca), (b_ref, ob_ref, cb)):
            x = x_ref[...]
            s = jnp.sum(x, axis=0)
            q = jnp.sum(x * x, axis=0)
            acc = jnp.concatenate(
                [s[None], q[None], jnp.zeros((6, c), F32)], axis=0)

            @pl.when(i == 0)
            def _():
                o_ref[...] = acc

            @pl.when(i > 0)
            def _():
                o_ref[...] = o_ref[...] + acc

    return pl.pallas_call(
        body,
        grid=(grid,),
        in_specs=[pl.BlockSpec((rb, ca), lambda i: (i, 0)),
                  pl.BlockSpec((rb, cb), lambda i: (i, 0))],
        out_specs=[pl.BlockSpec((8, ca), lambda i: (0, 0)),
                   pl.BlockSpec((8, cb), lambda i: (0, 0))],
        out_shape=[jax.ShapeDtypeStruct((8, ca), F32),
                   jax.ShapeDtypeStruct((8, cb), F32)],
    )(a, b)


def _stats1(a, n, rb):
    grid = n // rb
    ca = a.shape[1]

    def body(a_ref, oa_ref):
        i = pl.program_id(0)
        x = a_ref[...]
        s = jnp.sum(x, axis=0)
        q = jnp.sum(x * x, axis=0)
        acc = jnp.concatenate(
            [s[None], q[None], jnp.zeros((6, ca), F32)], axis=0)

        @pl.when(i == 0)
        def _():
            oa_ref[...] = acc

        @pl.when(i > 0)
        def _():
            oa_ref[...] = oa_ref[...] + acc

    return pl.pallas_call(
        body,
        grid=(grid,),
        in_specs=[pl.BlockSpec((rb, ca), lambda i: (i, 0))],
        out_specs=[pl.BlockSpec((8, ca), lambda i: (0, 0))],
        out_shape=[jax.ShapeDtypeStruct((8, ca), F32)],
    )(a)[0]


def _res_half1(a, b, sa, sb, emb, prm, ca, cb, nf, n, rb):
    grid = n // rb
    g1a = prm['g1'][:ca].reshape(1, ca)
    g1b = prm['g1'][ca:].reshape(1, cb)
    b1a = prm['b1'][:ca].reshape(1, ca)
    b1b = prm['b1'][ca:].reshape(1, cb)
    w1a = prm['W1'][:ca]
    w1b = prm['W1'][ca:]
    c1 = prm['c1'].reshape(1, nf)
    ce = prm['ce'].reshape(1, nf)

    def body(a_ref, b_ref, sa_ref, sb_ref, emb_ref, we_ref, g1a_ref,
             b1a_ref, g1b_ref, b1b_ref, w1a_ref, w1b_ref, c1_ref, ce_ref,
             mid_ref, ms_ref):
        i = pl.program_id(0)
        sca, ta = _bn_affine(sa_ref[...], g1a_ref[...], b1a_ref[...], n)
        scb, tb = _bn_affine(sb_ref[...], g1b_ref[...], b1b_ref[...], n)
        xa = _silu(a_ref[...] * sca + ta)
        xb = _silu(b_ref[...] * scb + tb)
        et = jnp.dot(_silu(emb_ref[...]), we_ref[...],
                     preferred_element_type=F32) + ce_ref[...]
        h = (jnp.dot(xa, w1a_ref[...], preferred_element_type=F32)
             + jnp.dot(xb, w1b_ref[...], preferred_element_type=F32)
             + c1_ref[...] + et)
        mid_ref[...] = h
        s = jnp.sum(h, axis=0)
        q = jnp.sum(h * h, axis=0)
        acc = jnp.concatenate([s[None], q[None], jnp.zeros((6, nf), F32)],
                              axis=0)

        @pl.when(i == 0)
        def _():
            ms_ref[...] = acc

        @pl.when(i > 0)
        def _():
            ms_ref[...] = ms_ref[...] + acc

    full = lambda r, c: pl.BlockSpec((r, c), lambda i: (0, 0))
    return pl.pallas_call(
        body,
        grid=(grid,),
        in_specs=[pl.BlockSpec((rb, ca), lambda i: (i, 0)),
                  pl.BlockSpec((rb, cb), lambda i: (i, 0)),
                  full(8, ca), full(8, cb), full(1, 512), full(512, nf),
                  full(1, ca), full(1, ca), full(1, cb), full(1, cb),
                  full(ca, nf), full(cb, nf), full(1, nf), full(1, nf)],
        out_specs=[pl.BlockSpec((rb, nf), lambda i: (i, 0)),
                   pl.BlockSpec((8, nf), lambda i: (0, 0))],
        out_shape=[jax.ShapeDtypeStruct((n, nf), F32),
                   jax.ShapeDtypeStruct((8, nf), F32)],
    )(a, b, sa, sb, emb, prm['We'], g1a, b1a, g1b, b1b, w1a, w1b, c1, ce)


def _res_half2(mid, ms, a, b, prm, up_w, up_b, ca, cb, nf, fo, n, rb):
    grid = n // rb
    g2 = prm['g2'].reshape(1, nf)
    b2 = prm['b2'].reshape(1, nf)
    c2 = prm['c2'].reshape(1, nf)
    wsa = prm['Ws'][:ca]
    wsb = prm['Ws'][ca:]
    upb = up_b.reshape(1, fo)

    def body(mid_ref, ms_ref, a_ref, b_ref, g2_ref, b2_ref, w2_ref,
             c2_ref, wsa_ref, wsb_ref, upw_ref, upb_ref, u_ref):
        sc, t = _bn_affine(ms_ref[...], g2_ref[...], b2_ref[...], n)
        x = _silu(mid_ref[...] * sc + t)
        h = (jnp.dot(x, w2_ref[...], preferred_element_type=F32)
             + jnp.dot(a_ref[...], wsa_ref[...], preferred_element_type=F32)
             + jnp.dot(b_ref[...], wsb_ref[...], preferred_element_type=F32)
             + c2_ref[...])
        u_ref[...] = jnp.dot(h, upw_ref[...],
                             preferred_element_type=F32) + upb_ref[...]

    full = lambda r, c: pl.BlockSpec((r, c), lambda i: (0, 0))
    return pl.pallas_call(
        body,
        grid=(grid,),
        in_specs=[pl.BlockSpec((rb, nf), lambda i: (i, 0)),
                  full(8, nf),
                  pl.BlockSpec((rb, ca), lambda i: (i, 0)),
                  pl.BlockSpec((rb, cb), lambda i: (i, 0)),
                  full(1, nf), full(1, nf), full(nf, nf), full(1, nf),
                  full(ca, nf), full(cb, nf), full(nf, fo), full(1, fo)],
        out_specs=[pl.BlockSpec((rb, fo), lambda i: (i, 0))],
        out_shape=[jax.ShapeDtypeStruct((n, fo), F32)],
    )(mid, ms, a, b, g2, b2, prm['W2'], c2, wsa, wsb, up_w, upb)[0]


def _point_mlp(z, sz, g, b, w, c, n_stats, rb):
    n = z.shape[0]
    grid = n // rb
    cz, fo = w.shape
    g2 = g.reshape(1, cz)
    b2 = b.reshape(1, cz)
    c2 = c.reshape(1, fo)

    def body(z_ref, sz_ref, g_ref, b_ref, w_ref, c_ref, o_ref):
        sc, t = _bn_affine(sz_ref[...], g_ref[...], b_ref[...], n_stats)
        x = _silu(z_ref[...] * sc + t)
        o_ref[...] = jnp.dot(x, w_ref[...],
                             preferred_element_type=F32) + c_ref[...]

    full = lambda r, cc: pl.BlockSpec((r, cc), lambda i: (0, 0))
    return pl.pallas_call(
        body,
        grid=(grid,),
        in_specs=[pl.BlockSpec((rb, cz), lambda i: (i, 0)),
                  full(8, cz), full(1, cz), full(1, cz),
                  full(cz, fo), full(1, fo)],
        out_specs=[pl.BlockSpec((rb, fo), lambda i: (i, 0))],
        out_shape=[jax.ShapeDtypeStruct((n, fo), F32)],
    )(z, sz, g2, b2, w, c2)[0]


def _seg_divide(sums, cnts, n, rb):
    grid = n // rb
    fo = sums.shape[1]

    def body(s_ref, c_ref, o_ref):
        o_ref[...] = s_ref[...] / jnp.maximum(c_ref[...], 1.0)

    return pl.pallas_call(
        body,
        grid=(grid,),
        in_specs=[pl.BlockSpec((rb, fo), lambda i: (i, 0)),
                  pl.BlockSpec((rb, 1), lambda i: (i, 0))],
        out_specs=[pl.BlockSpec((rb, fo), lambda i: (i, 0))],
        out_shape=[jax.ShapeDtypeStruct((n, fo), F32)],
    )(sums, cnts)[0]


def _sc_mesh():
    return plsc.VectorSubcoreMesh(core_axis_name="c", subcore_axis_name="s")


def _gather_rows(table, idx, d):
    b = idx.shape[0]
    per_w = b // NW
    n_ch = per_w // CH

    @functools.partial(
        pl.kernel, mesh=_sc_mesh(),
        compiler_params=pltpu.CompilerParams(use_tc_tiling_on_sc=False, needs_layout_passes=False),
        out_type=jax.ShapeDtypeStruct((b, d), F32),
        scratch_types=[
            pltpu.VMEM((per_w,), I32),
            pltpu.VMEM((per_w, d), F32),
            pltpu.SemaphoreType.DMA,
        ],
    )
    def k(tab_hbm, idx_hbm, out_hbm, idx_v, rows_v, sem):
        wid = lax.axis_index("s") * NC + lax.axis_index("c")
        base = wid * per_w
        pltpu.sync_copy(idx_hbm.at[pl.ds(base, per_w)], idx_v)
        hs = [pltpu.async_copy(tab_hbm.at[idx_v.at[pl.ds(j * CH, CH)]],
                               rows_v.at[pl.ds(j * CH, CH)], sem)
              for j in range(n_ch)]
        for h in hs:
            h.wait()
        pltpu.sync_copy(rows_v, out_hbm.at[pl.ds(base, per_w)])

    return k(table, idx)


def _point_gather(u1, idx_p2v_p, idx_up1_p, pb):
    fo = u1.shape[1]
    per_w = NPP // NW
    nb = per_w // KCH
    seg = NPP // NS

    @functools.partial(
        pl.kernel, mesh=_sc_mesh(),
        compiler_params=pltpu.CompilerParams(use_tc_tiling_on_sc=False, needs_layout_passes=False),
        out_type=jax.ShapeDtypeStruct((NPP, fo), F32),
        scratch_types=[
            pltpu.VMEM_SHARED((NPP,), I32),
            pltpu.VMEM((2, KCH), I32),
            pltpu.VMEM((2, KCH, fo), F32),
            pltpu.VMEM((KCH,), I32),
            pltpu.VMEM((KCH, fo), F32),
            pltpu.VMEM((seg,), I32),
            pltpu.SemaphoreType.DMA,
            pltpu.SemaphoreType.DMA,
            pltpu.SemaphoreType.DMA,
        ],
    )
    def k(u1_hbm, idxp_hbm, tab_hbm, pb_hbm, z1_hbm,
          tab_sp, idxp_v, pb_v, idxc_v, z_v, tab_stage,
          sem_in, sem_c, sem_g):
        c = lax.axis_index("c")
        s = lax.axis_index("s")
        wid = s * NC + c
        base0 = wid * per_w

        pltpu.sync_copy(tab_hbm.at[pl.ds(s * seg, seg)], tab_stage)
        pltpu.sync_copy(tab_stage, tab_sp.at[pl.ds(s * seg, seg)])
        plsc.subcore_barrier()

        def fire(b, slot):
            base = base0 + b * KCH
            return (pltpu.async_copy(idxp_hbm.at[pl.ds(base, KCH)],
                                     idxp_v.at[slot], sem_in),
                    pltpu.async_copy(pb_hbm.at[pl.ds(base, KCH)],
                                     pb_v.at[slot], sem_in))

        hs = [None, None]
        hs[0] = fire(0, 0)
        for b in range(nb):
            sl = b % 2
            if b + 1 < nb:
                hs[(b + 1) % 2] = fire(b + 1, (b + 1) % 2)
            h_ip, h_pb = hs[sl]
            h_ip.wait()
            hc = [pltpu.async_copy(
                      tab_sp.at[idxp_v.at[sl].at[pl.ds(kk * CH, CH)]],
                      idxc_v.at[pl.ds(kk * CH, CH)], sem_c)
                  for kk in range(KB)]
            for h in hc:
                h.wait()
            hg = [pltpu.async_copy(
                      u1_hbm.at[idxc_v.at[pl.ds(kk * CH, CH)]],
                      z_v.at[pl.ds(kk * CH, CH)], sem_g)
                  for kk in range(KB)]
            h_pb.wait()
            for h in hg:
                h.wait()

            @plsc.parallel_loop(0, KCH, unroll=4)
            def _ad(r, sl=sl):
                z_v[r, pl.ds(0, L)] = (z_v[r, pl.ds(0, L)]
                                       + pb_v[sl, r, pl.ds(0, L)])
                z_v[r, pl.ds(L, L)] = (z_v[r, pl.ds(L, L)]
                                       + pb_v[sl, r, pl.ds(L, L)])

            pltpu.sync_copy(z_v, z1_hbm.at[pl.ds(base0 + b * KCH, KCH)])

    return k(u1, idx_p2v_p, idx_up1_p, pb)


def _point_stage(u1, idx_p2v_p, idx_up1_p, pb):
    fo = u1.shape[1]
    n_ch = NPP // CH
    per_tile = n_ch // NS
    KB = 2
    KCH = KB * CH
    nb = per_tile // KB
    drain_ch = HALF // CH
    zero_ch = ACC_R // CH
    seg = NPP // NS

    @functools.partial(
        pl.kernel, mesh=_sc_mesh(),
        compiler_params=pltpu.CompilerParams(use_tc_tiling_on_sc=False, needs_layout_passes=False),
        out_type=(jax.ShapeDtypeStruct((NPP, fo), F32),
                  jax.ShapeDtypeStruct((NPP, fo), F32),
                  jax.ShapeDtypeStruct((NPP,), F32)),
        scratch_types=[
            pltpu.VMEM_SHARED((ACC_R, fo), F32),
            pltpu.VMEM_SHARED((ACC_R,), F32),
            pltpu.VMEM_SHARED((NPP,), I32),
            pltpu.VMEM((KCH, fo), F32),
            pltpu.VMEM((KCH, fo), F32),
            pltpu.VMEM((KCH,), I32),
            pltpu.VMEM((KCH,), I32),
            pltpu.VMEM((KB, CH), I32),
            pltpu.VMEM((CH,), F32),
            pltpu.VMEM((CH,), F32),
            pltpu.SemaphoreType.DMA,
            pltpu.SemaphoreType.DMA,
            pltpu.SemaphoreType.DMA,
        ],
    )
    def k(u1_hbm, idxp_hbm, tab_hbm, pb_hbm,
          z1_hbm, sums_hbm, cnts_hbm,
          acc, acc1, tab_sp, z_v, pb_v, idxp_v, idxc_v, sidx_v,
          ones_v, zf_v, sem_in, sem_c, sem_g):
        c = lax.axis_index("c")
        s = lax.axis_index("s")
        lo = c * HALF

        @plsc.parallel_loop(0, CH // L)
        def _i1(g):
            ones_v[pl.ds(g * L, L)] = jnp.ones((L,), F32)
            zf_v[pl.ds(g * L, L)] = jnp.zeros((L,), F32)

        @plsc.parallel_loop(0, CH)
        def _i2(r):
            pb_v[r, pl.ds(0, L)] = jnp.zeros((L,), F32)
            pb_v[r, pl.ds(L, L)] = jnp.zeros((L,), F32)

        pltpu.sync_copy(tab_hbm.at[pl.ds(s * seg, seg)],
                        tab_sp.at[pl.ds(s * seg, seg)])

        def zacc(j, cr):
            ch = s + j * NS

            @pl.when(ch < zero_ch)
            def _w():
                pltpu.sync_copy(pb_v.at[pl.ds(0, CH)],
                                acc.at[pl.ds(ch * CH, CH)])
                pltpu.sync_copy(zf_v, acc1.at[pl.ds(ch * CH, CH)])
            return cr
        lax.fori_loop(0, (zero_ch + NS - 1) // NS, zacc, 0)

        plsc.subcore_barrier()

        for b in range(nb):
            base = (s * per_tile + b * KB) * CH
            h_ip = pltpu.async_copy(idxp_hbm.at[pl.ds(base, KCH)],
                                    idxp_v, sem_in)
            h_pb = pltpu.async_copy(pb_hbm.at[pl.ds(base, KCH)],
                                    pb_v, sem_in)
            h_ip.wait()
            hc = [pltpu.async_copy(
                      tab_sp.at[idxp_v.at[pl.ds(kk * CH, CH)]],
                      idxc_v.at[pl.ds(kk * CH, CH)], sem_c)
                  for kk in range(KB)]

            for kk in range(KB):
                @plsc.parallel_loop(0, CH // L)
                def _sx(g, kk=kk):
                    iv = idxp_v[pl.ds(kk * CH + g * L, L)]
                    inr = jnp.logical_and(iv >= lo, iv < lo + HALF)
                    sidx_v[kk, pl.ds(g * L, L)] = jnp.where(
                        inr, iv - lo, DUMP)

            for h in hc:
                h.wait()
            hg = [pltpu.async_copy(
                      u1_hbm.at[idxc_v.at[pl.ds(kk * CH, CH)]],
                      z_v.at[pl.ds(kk * CH, CH)], sem_g)
                  for kk in range(KB)]
            h_pb.wait()
            for h in hg:
                h.wait()

            @plsc.parallel_loop(0, KCH, unroll=4)
            def _ad(r):
                z_v[r, pl.ds(0, L)] = z_v[r, pl.ds(0, L)] + pb_v[r, pl.ds(0, L)]
                z_v[r, pl.ds(L, L)] = z_v[r, pl.ds(L, L)] + pb_v[r, pl.ds(L, L)]

            for kk in range(KB):
                @pl.when(jnp.int32((b * KB + kk) % 2) == c)
                def _w(kk=kk):
                    pltpu.sync_copy(z_v.at[pl.ds(kk * CH, CH)],
                                    z1_hbm.at[pl.ds(base + kk * CH, CH)])
                pltpu.sync_copy(z_v.at[pl.ds(kk * CH, CH)],
                                acc.at[sidx_v.at[kk]], add=True)
                pltpu.sync_copy(ones_v, acc1.at[sidx_v.at[kk]], add=True)

        plsc.subcore_barrier()

        def drain(j, _):
            ch = s + j * NS
            row = ch * CH
            pltpu.sync_copy(acc.at[pl.ds(row, CH)], z_v.at[pl.ds(0, CH)])
            pltpu.sync_copy(z_v.at[pl.ds(0, CH)],
                            sums_hbm.at[pl.ds(lo + row, CH)])
            pltpu.sync_copy(acc1.at[pl.ds(row, CH)], ones_v)
            pltpu.sync_copy(ones_v, cnts_hbm.at[pl.ds(lo + row, CH)])
            return _
        lax.fori_loop(0, drain_ch // NS, drain, 0)

    return k(u1, idx_p2v_p, idx_up1_p, pb)


def kernel(x_feat, z_feat, emb, skip0, skip1, params, idx_up0, idx_up1,
           idx_p2v):
    p = params
    idx_up0_p = jnp.concatenate(
        [idx_up0.astype(I32), jnp.zeros((NV1P - NV1,), I32)])
    idx_up1_p = jnp.concatenate(
        [idx_up1.astype(I32), jnp.zeros((NPP - NV2,), I32)])
    idx_p2v_p = jnp.concatenate(
        [idx_p2v.astype(I32), jnp.full((NPP - NP_,), NV2, I32)])
    z_feat_p = jnp.concatenate(
        [z_feat, jnp.zeros((NPP - NP_, z_feat.shape[1]), F32)])

    sa0, sb0 = _stats2(x_feat, skip0, NV0, 5000)
    mid0, ms0 = _res_half1(x_feat, skip0, sa0, sb0, emb, p['res0'],
                           128, 64, 128, NV0, 5000)
    u0 = _res_half2(mid0, ms0, x_feat, skip0, p['res0'],
                    p['up0_W'], p['up0_b'], 128, 64, 128, 64, NV0, 5000)
    x1 = _gather_rows(u0, idx_up0_p, 64)

    sa1, sb1 = _stats2(x1, skip1, NV1, 5000)
    mid1, ms1 = _res_half1(x1, skip1, sa1, sb1, emb, p['res1'],
                           64, 32, 64, NV1, 5000)
    u1 = _res_half2(mid1, ms1, x1, skip1, p['res1'],
                    p['up1_W'], p['up1_b'], 64, 32, 64, 32, NV1, 5000)

    sz = _stats1(z_feat, NP_, 5000)
    pb = _point_mlp(z_feat_p, sz, p['pb_g'], p['pb_b'], p['pb_W'],
                    p['pb_c'], NP_, 5120)

    z1p, sums, cnts = _point_stage(u1, idx_p2v_p, idx_up1_p, pb)

    x_out = _seg_divide(sums, cnts.reshape(NPP, 1), NV2, 5000)
    return (x_out, z1p[:NP_])
